# Initial kernel scaffold; baseline (speedup 1.0000x reference)
import jax
import jax.numpy as jnp
from jax.experimental import pallas as pl


def kernel(x, edge_index, fc1_w, fc1_b, attn1_w, attn2_w, fc2_w, fc2_b):
    raise NotImplementedError("write your pallas kernel here")


# speedup vs baseline: 8.4483x; 8.4483x over previous
import functools

import jax
import jax.numpy as jnp
from jax import lax
from jax.experimental import pallas as pl
from jax.experimental.pallas import tpu as pltpu
from jax.experimental.pallas import tpu_sc as plsc

N = 10000
D = 128
E = 320000
HOP = 10
NC = 2
NS = 16
NW = NC * NS
EPT = E // NW
CHUNK = 80
NCH = EPT // CHUNK
NP = 10240
NPT = NP // NS
BLK = 2000


def _leaky_exp(v):
    return jnp.exp(jnp.where(v >= 0.0, v, 0.2 * v))


def _pre_body(x_ref, w1_ref, b1_ref, a1_ref, a2_ref,
              h_ref, x1_ref, xa2_ref, w2_ref, wx_ref):
    xb = x_ref[...]
    hb = jnp.maximum(xb @ w1_ref[...].T + b1_ref[...][None, :], 0.0)
    x1 = hb @ a1_ref[...].T
    xa2 = hb @ a2_ref[...].T
    w2 = _leaky_exp(x1 + xa2)
    h_ref[...] = hb
    x1_ref[...] = x1
    xa2_ref[...] = xa2
    w2_ref[...] = w2
    wx_ref[...] = w2 * hb


def _combine_body(acc_ref, dacc_ref, wx_ref, w2_ref, a2_ref, h_ref, h1_ref):
    num = acc_ref[0] + acc_ref[1] + wx_ref[...]
    den = dacc_ref[0] + dacc_ref[1] + w2_ref[...]
    hv = num / den
    hv = jnp.where(hv > 0.0, hv, jnp.exp(hv) - 1.0)
    h_ref[...] = hv
    h1_ref[...] = hv @ a2_ref[...].T


def _post_body(h_ref, w_ref, b_ref, o_ref):
    o_ref[...] = h_ref[...] @ w_ref[...].T + b_ref[...][None, :]


def _pre(x, fc1_w, fc1_b, attn1_w, attn2_w):
    g = N // BLK
    return pl.pallas_call(
        _pre_body,
        grid=(g,),
        in_specs=[
            pl.BlockSpec((BLK, D), lambda i: (i, 0)),
            pl.BlockSpec((D, D), lambda i: (0, 0)),
            pl.BlockSpec((D,), lambda i: (0,)),
            pl.BlockSpec((1, D), lambda i: (0, 0)),
            pl.BlockSpec((1, D), lambda i: (0, 0)),
        ],
        out_specs=[
            pl.BlockSpec((BLK, D), lambda i: (i, 0)),
            pl.BlockSpec((BLK, 1), lambda i: (i, 0)),
            pl.BlockSpec((BLK, 1), lambda i: (i, 0)),
            pl.BlockSpec((BLK, 1), lambda i: (i, 0)),
            pl.BlockSpec((BLK, D), lambda i: (i, 0)),
        ],
        out_shape=[
            jax.ShapeDtypeStruct((N, D), jnp.float32),
            jax.ShapeDtypeStruct((N, 1), jnp.float32),
            jax.ShapeDtypeStruct((N, 1), jnp.float32),
            jax.ShapeDtypeStruct((N, 1), jnp.float32),
            jax.ShapeDtypeStruct((N, D), jnp.float32),
        ],
    )(x, fc1_w, fc1_b, attn1_w, attn2_w)


def _combine(acc, dacc3, wx, w2, attn2_w):
    g = N // BLK
    return pl.pallas_call(
        _combine_body,
        grid=(g,),
        in_specs=[
            pl.BlockSpec((NC, BLK, D), lambda i: (0, i, 0)),
            pl.BlockSpec((NC, BLK, 1), lambda i: (0, i, 0)),
            pl.BlockSpec((BLK, D), lambda i: (i, 0)),
            pl.BlockSpec((BLK, 1), lambda i: (i, 0)),
            pl.BlockSpec((1, D), lambda i: (0, 0)),
        ],
        out_specs=[
            pl.BlockSpec((BLK, D), lambda i: (i, 0)),
            pl.BlockSpec((BLK, 1), lambda i: (i, 0)),
        ],
        out_shape=[
            jax.ShapeDtypeStruct((N, D), jnp.float32),
            jax.ShapeDtypeStruct((N, 1), jnp.float32),
        ],
    )(acc, dacc3, wx, w2, attn2_w)


def _post(h, fc2_w, fc2_b):
    g = N // BLK
    return pl.pallas_call(
        _post_body,
        grid=(g,),
        in_specs=[
            pl.BlockSpec((BLK, D), lambda i: (i, 0)),
            pl.BlockSpec((D, D), lambda i: (0, 0)),
            pl.BlockSpec((D,), lambda i: (0,)),
        ],
        out_specs=pl.BlockSpec((BLK, D), lambda i: (i, 0)),
        out_shape=jax.ShapeDtypeStruct((N, D), jnp.float32),
    )(h, fc2_w, fc2_b)


def _sc_hop_body(h_hbm, h1_hbm, x1_hbm, s_hbm, t_hbm, z2_hbm, z1_hbm,
                 acc_hbm, dacc_hbm,
                 x1_l, h1_l, sbuf, tbuf, w1b, rows, acc_sh, div_sh, sem):
    cid = lax.axis_index("c")
    sid = lax.axis_index("s")
    wid = sid * NC + cid

    nsl = pl.ds(sid * NPT, NPT)
    pltpu.sync_copy(z2_hbm, acc_sh.at[nsl])
    pltpu.sync_copy(z1_hbm, div_sh.at[nsl])
    pltpu.sync_copy(x1_hbm, x1_l)
    pltpu.sync_copy(h1_hbm, h1_l)
    plsc.subcore_barrier()

    ebase = wid * EPT

    def chunk(ci, carry):
        eoff = ebase + ci * CHUNK
        pltpu.sync_copy(s_hbm.at[pl.ds(eoff, CHUNK)], sbuf)
        pltpu.sync_copy(t_hbm.at[pl.ds(eoff, CHUNK)], tbuf)
        cp = pltpu.async_copy(h_hbm.at[tbuf], rows, sem)
        for j in range(CHUNK // 16):
            sl = pl.ds(j * 16, 16)
            v = (plsc.load_gather(x1_l, [sbuf[sl]])
                 + plsc.load_gather(h1_l, [tbuf[sl]]))
            w1b[sl] = _leaky_exp(v)
        cp.wait()
        for j in range(CHUNK // 16):
            wv = w1b[pl.ds(j * 16, 16)]
            for k in range(16):
                w = wv[k]
                e = j * 16 + k
                for cc in range(D // 16):
                    csl = pl.ds(cc * 16, 16)
                    rows[e, csl] = rows[e, csl] * w
        pltpu.sync_copy(rows, acc_sh.at[sbuf], add=True)
        pltpu.sync_copy(w1b, div_sh.at[sbuf], add=True)
        return carry

    lax.fori_loop(0, NCH, chunk, 0)
    plsc.subcore_barrier()
    pltpu.sync_copy(acc_sh.at[nsl], acc_hbm.at[cid, nsl])
    pltpu.sync_copy(div_sh.at[nsl], dacc_hbm.at[cid, nsl])


def _sc_hop(h, h1f, x1f, s, t, z2, z1):
    mesh = plsc.VectorSubcoreMesh(core_axis_name="c", subcore_axis_name="s",
                                  num_cores=NC, num_subcores=NS)
    return pl.kernel(
        _sc_hop_body,
        out_type=(jax.ShapeDtypeStruct((NC, NP, D), jnp.float32),
                  jax.ShapeDtypeStruct((NC, NP), jnp.float32)),
        mesh=mesh,
        compiler_params=pltpu.CompilerParams(needs_layout_passes=False),
        scratch_types=[
            pltpu.VMEM((N,), jnp.float32),
            pltpu.VMEM((N,), jnp.float32),
            pltpu.VMEM((CHUNK,), jnp.int32),
            pltpu.VMEM((CHUNK,), jnp.int32),
            pltpu.VMEM((CHUNK,), jnp.float32),
            pltpu.VMEM((CHUNK, D), jnp.float32),
            pltpu.VMEM_SHARED((NP, D), jnp.float32),
            pltpu.VMEM_SHARED((NP,), jnp.float32),
            pltpu.SemaphoreType.DMA,
        ],
    )(h, h1f, x1f, s, t, z2, z1)


def kernel(x, edge_index, fc1_w, fc1_b, attn1_w, attn2_w, fc2_w, fc2_b):
    s = edge_index[0]
    t = edge_index[1]
    h, x1, xa2, w2, wx = _pre(x, fc1_w, fc1_b, attn1_w, attn2_w)
    x1f = x1[:, 0]
    h1f = xa2[:, 0]
    z2 = jnp.zeros((NPT, D), jnp.float32)
    z1 = jnp.zeros((NPT,), jnp.float32)
    for _ in range(HOP):
        acc, dacc = _sc_hop(h, h1f, x1f, s, t, z2, z1)
        h, h1 = _combine(acc, dacc[:, :, None], wx, w2, attn2_w)
        h1f = h1[:, 0]
    return _post(h, fc2_w, fc2_b)


# speedup vs baseline: 12.4799x; 1.4772x over previous
import functools

import jax
import jax.numpy as jnp
from jax import lax
from jax.experimental import pallas as pl
from jax.experimental.pallas import tpu as pltpu
from jax.experimental.pallas import tpu_sc as plsc

N = 10000
D = 128
E = 320000
HOP = 10
NC = 2
NS = 16
NW = NC * NS
EPT = 10240
E2 = NW * EPT
CHUNK = 80
NCH = EPT // CHUNK
NP = 10240
NPT = NP // NS
BLK = 2000


def _leaky_exp(v):
    return jnp.exp(jnp.where(v >= 0.0, v, 0.2 * v))


def _pre_body(x_ref, w1_ref, b1_ref, a1_ref, a2_ref,
              h_ref, x1_ref, xa2_ref, w2_ref, wx_ref):
    xb = x_ref[...]
    hb = jnp.maximum(xb @ w1_ref[...].T + b1_ref[...][None, :], 0.0)
    x1 = hb @ a1_ref[...].T
    xa2 = hb @ a2_ref[...].T
    w2 = _leaky_exp(x1 + xa2)
    h_ref[...] = hb
    x1_ref[...] = x1
    xa2_ref[...] = xa2
    w2_ref[...] = w2
    wx_ref[...] = w2 * hb


def _combine_body(acc_ref, dacc_ref, wx_ref, w2_ref, a2_ref, h_ref, h1_ref):
    num = acc_ref[0] + acc_ref[1] + wx_ref[...]
    den = dacc_ref[0] + dacc_ref[1] + w2_ref[...]
    hv = num / den
    hv = jnp.where(hv > 0.0, hv, jnp.exp(hv) - 1.0)
    h_ref[...] = hv
    h1_ref[...] = hv @ a2_ref[...].T


def _post_body(h_ref, w_ref, b_ref, o_ref):
    o_ref[...] = h_ref[...] @ w_ref[...].T + b_ref[...][None, :]


def _pre(x, fc1_w, fc1_b, attn1_w, attn2_w):
    g = N // BLK
    return pl.pallas_call(
        _pre_body,
        grid=(g,),
        in_specs=[
            pl.BlockSpec((BLK, D), lambda i: (i, 0)),
            pl.BlockSpec((D, D), lambda i: (0, 0)),
            pl.BlockSpec((D,), lambda i: (0,)),
            pl.BlockSpec((1, D), lambda i: (0, 0)),
            pl.BlockSpec((1, D), lambda i: (0, 0)),
        ],
        out_specs=[
            pl.BlockSpec((BLK, D), lambda i: (i, 0)),
            pl.BlockSpec((BLK, 1), lambda i: (i, 0)),
            pl.BlockSpec((BLK, 1), lambda i: (i, 0)),
            pl.BlockSpec((BLK, 1), lambda i: (i, 0)),
            pl.BlockSpec((BLK, D), lambda i: (i, 0)),
        ],
        out_shape=[
            jax.ShapeDtypeStruct((N, D), jnp.float32),
            jax.ShapeDtypeStruct((N, 1), jnp.float32),
            jax.ShapeDtypeStruct((N, 1), jnp.float32),
            jax.ShapeDtypeStruct((N, 1), jnp.float32),
            jax.ShapeDtypeStruct((N, D), jnp.float32),
        ],
    )(x, fc1_w, fc1_b, attn1_w, attn2_w)


def _combine(acc, dacc3, wx, w2, attn2_w):
    g = N // BLK
    return pl.pallas_call(
        _combine_body,
        grid=(g,),
        in_specs=[
            pl.BlockSpec((NC, BLK, D), lambda i: (0, i, 0)),
            pl.BlockSpec((NC, BLK, 1), lambda i: (0, i, 0)),
            pl.BlockSpec((BLK, D), lambda i: (i, 0)),
            pl.BlockSpec((BLK, 1), lambda i: (i, 0)),
            pl.BlockSpec((1, D), lambda i: (0, 0)),
        ],
        out_specs=[
            pl.BlockSpec((BLK, D), lambda i: (i, 0)),
            pl.BlockSpec((BLK, 1), lambda i: (i, 0)),
        ],
        out_shape=[
            jax.ShapeDtypeStruct((N, D), jnp.float32),
            jax.ShapeDtypeStruct((N, 1), jnp.float32),
        ],
    )(acc, dacc3, wx, w2, attn2_w)


def _post(h, fc2_w, fc2_b):
    g = N // BLK
    return pl.pallas_call(
        _post_body,
        grid=(g,),
        in_specs=[
            pl.BlockSpec((BLK, D), lambda i: (i, 0)),
            pl.BlockSpec((D, D), lambda i: (0, 0)),
            pl.BlockSpec((D,), lambda i: (0,)),
        ],
        out_specs=pl.BlockSpec((BLK, D), lambda i: (i, 0)),
        out_shape=jax.ShapeDtypeStruct((N, D), jnp.float32),
    )(h, fc2_w, fc2_b)


def _sc_hop_body(h_hbm, h1_hbm, x1_hbm, s_hbm, t_hbm, z2_hbm, z1_hbm,
                 acc_hbm, dacc_hbm,
                 x1_l, h1_l, sbufA, tbufA, sbufB, tbufB, sidxA, sidxB,
                 w1A, w1B, rowsA, rowsB, acc_sh, div_sh,
                 semGA, semGB, semSA, semSB, semIA, semIB):
    cid = lax.axis_index("c")
    sid = lax.axis_index("s")
    wid = sid * NC + cid

    nsl = pl.ds(sid * NPT, NPT)
    pltpu.sync_copy(z2_hbm, acc_sh.at[nsl])
    pltpu.sync_copy(z1_hbm, div_sh.at[nsl])
    pltpu.sync_copy(x1_hbm, x1_l)
    pltpu.sync_copy(h1_hbm, h1_l)
    plsc.subcore_barrier()

    ebase = wid * EPT

    def idx_load(ci, sbuf, tbuf, sem):
        eoff = ebase + ci * CHUNK
        c1 = pltpu.async_copy(s_hbm.at[pl.ds(eoff, CHUNK)], sbuf, sem)
        c2 = pltpu.async_copy(t_hbm.at[pl.ds(eoff, CHUNK)], tbuf, sem)
        return c1, c2

    def idx_wait(sbuf, tbuf, sem):
        pltpu.make_async_copy(s_hbm.at[pl.ds(0, CHUNK)], sbuf, sem).wait()
        pltpu.make_async_copy(t_hbm.at[pl.ds(0, CHUNK)], tbuf, sem).wait()

    def gather(rows, tbuf, sem):
        return pltpu.async_copy(h_hbm.at[tbuf], rows, sem)

    def gather_wait(rows, tbuf, sem):
        pltpu.make_async_copy(h_hbm.at[tbuf], rows, sem).wait()

    def process(sbuf, tbuf, sidx, rows, w1b):
        for j in range(CHUNK // 16):
            sl = pl.ds(j * 16, 16)
            sv = sbuf[sl]
            v = (plsc.load_gather(x1_l, [sv])
                 + plsc.load_gather(h1_l, [tbuf[sl]]))
            w1 = _leaky_exp(v)
            w1b[sl] = w1
            sidx[sl] = sv
            for k in range(16):
                w = w1[k]
                e = j * 16 + k
                for cc in range(D // 16):
                    csl = pl.ds(cc * 16, 16)
                    rows[e, csl] = rows[e, csl] * w

    def scatter(rows, w1b, sidx, sem):
        pltpu.async_copy(rows, acc_sh.at[sidx], sem, add=True)
        pltpu.async_copy(w1b, div_sh.at[sidx], sem, add=True)

    def scatter_wait(rows, w1b, sidx, sem):
        pltpu.make_async_copy(rows, acc_sh.at[sidx], sem).wait()
        pltpu.make_async_copy(w1b, div_sh.at[sidx], sem).wait()

    idx_load(0, sbufA, tbufA, semIA)
    idx_load(1, sbufB, tbufB, semIB)
    idx_wait(sbufA, tbufA, semIA)
    idx_wait(sbufB, tbufB, semIB)
    gather(rowsA, tbufA, semGA)
    gather(rowsB, tbufB, semGB)

    def body(p, carry):
        c0 = 2 * p
        gather_wait(rowsA, tbufA, semGA)
        process(sbufA, tbufA, sidxA, rowsA, w1A)
        scatter(rowsA, w1A, sidxA, semSA)
        idx_load(c0 + 2, sbufA, tbufA, semIA)
        gather_wait(rowsB, tbufB, semGB)
        process(sbufB, tbufB, sidxB, rowsB, w1B)
        scatter(rowsB, w1B, sidxB, semSB)
        idx_load(c0 + 3, sbufB, tbufB, semIB)
        scatter_wait(rowsA, w1A, sidxA, semSA)
        idx_wait(sbufA, tbufA, semIA)
        gather(rowsA, tbufA, semGA)
        scatter_wait(rowsB, w1B, sidxB, semSB)
        idx_wait(sbufB, tbufB, semIB)
        gather(rowsB, tbufB, semGB)
        return carry

    lax.fori_loop(0, NCH // 2 - 1, body, 0)
    gather_wait(rowsA, tbufA, semGA)
    process(sbufA, tbufA, sidxA, rowsA, w1A)
    scatter(rowsA, w1A, sidxA, semSA)
    gather_wait(rowsB, tbufB, semGB)
    process(sbufB, tbufB, sidxB, rowsB, w1B)
    scatter(rowsB, w1B, sidxB, semSB)
    scatter_wait(rowsA, w1A, sidxA, semSA)
    scatter_wait(rowsB, w1B, sidxB, semSB)

    plsc.subcore_barrier()
    pltpu.sync_copy(acc_sh.at[nsl], acc_hbm.at[cid, nsl])
    pltpu.sync_copy(div_sh.at[nsl], dacc_hbm.at[cid, nsl])


def _sc_hop(h, h1p, x1p, s2, t2, z2, z1):
    mesh = plsc.VectorSubcoreMesh(core_axis_name="c", subcore_axis_name="s",
                                  num_cores=NC, num_subcores=NS)
    return pl.kernel(
        _sc_hop_body,
        out_type=(jax.ShapeDtypeStruct((NC, NP, D), jnp.float32),
                  jax.ShapeDtypeStruct((NC, NP), jnp.float32)),
        mesh=mesh,
        compiler_params=pltpu.CompilerParams(needs_layout_passes=False),
        scratch_types=[
            pltpu.VMEM((NP,), jnp.float32),
            pltpu.VMEM((NP,), jnp.float32),
            pltpu.VMEM((CHUNK,), jnp.int32),
            pltpu.VMEM((CHUNK,), jnp.int32),
            pltpu.VMEM((CHUNK,), jnp.int32),
            pltpu.VMEM((CHUNK,), jnp.int32),
            pltpu.VMEM((CHUNK,), jnp.int32),
            pltpu.VMEM((CHUNK,), jnp.int32),
            pltpu.VMEM((CHUNK,), jnp.float32),
            pltpu.VMEM((CHUNK,), jnp.float32),
            pltpu.VMEM((CHUNK, D), jnp.float32),
            pltpu.VMEM((CHUNK, D), jnp.float32),
            pltpu.VMEM_SHARED((NP, D), jnp.float32),
            pltpu.VMEM_SHARED((NP,), jnp.float32),
            pltpu.SemaphoreType.DMA,
            pltpu.SemaphoreType.DMA,
            pltpu.SemaphoreType.DMA,
            pltpu.SemaphoreType.DMA,
            pltpu.SemaphoreType.DMA,
            pltpu.SemaphoreType.DMA,
        ],
    )(h, h1p, x1p, s2, t2, z2, z1)


def kernel(x, edge_index, fc1_w, fc1_b, attn1_w, attn2_w, fc2_w, fc2_b):
    npad = E2 - E
    s2 = jnp.concatenate([edge_index[0],
                          N + (jnp.arange(npad, dtype=jnp.int32) % (NP - N))])
    t2 = jnp.concatenate([edge_index[1],
                          jnp.arange(npad, dtype=jnp.int32) % N])
    h, x1, xa2, w2, wx = _pre(x, fc1_w, fc1_b, attn1_w, attn2_w)
    zpad = jnp.zeros((NP - N,), jnp.float32)
    x1p = jnp.concatenate([x1[:, 0], zpad])
    h1p = jnp.concatenate([xa2[:, 0], zpad])
    z2 = jnp.zeros((NPT, D), jnp.float32)
    z1 = jnp.zeros((NPT,), jnp.float32)
    for _ in range(HOP):
        acc, dacc = _sc_hop(h, h1p, x1p, s2, t2, z2, z1)
        h, h1 = _combine(acc, dacc[:, :, None], wx, w2, attn2_w)
        h1p = jnp.concatenate([h1[:, 0], zpad])
    return _post(h, fc2_w, fc2_b)
